# Initial kernel scaffold; baseline (speedup 1.0000x reference)
import jax
import jax.numpy as jnp
from jax.experimental import pallas as pl


def kernel(x, embed, W1, b1, W2, b2):
    raise NotImplementedError("write your pallas kernel here")


# speedup vs baseline: 11.6614x; 11.6614x over previous
import jax
import jax.numpy as jnp
from jax.experimental import pallas as pl

CONTEXT = 16
VOCAB = 256
EMBED = 6
HIDDEN = 64
NOUT = 256
BB = 512


def _table_body(embed_ref, w1r_ref, b1_ref, tbl_ref):
    tbl_ref[0] = (
        jnp.dot(embed_ref[...], w1r_ref[0], preferred_element_type=jnp.float32)
        + b1_ref[...] / CONTEXT
    )


def _mlp_body(x_ref, tbl_ref, w2t_ref, b2_ref, out_ref):
    acc = jnp.zeros((BB, HIDDEN), dtype=jnp.float32)
    iota = jax.lax.broadcasted_iota(jnp.int32, (BB, VOCAB), 1)
    for t in range(CONTEXT):
        col = x_ref[:, t : t + 1]
        mh = (col == iota).astype(jnp.float32)
        acc = acc + jnp.dot(mh, tbl_ref[t], preferred_element_type=jnp.float32)
    h1 = jnp.maximum(acc, 0.0)
    out = jnp.dot(h1, w2t_ref[...], preferred_element_type=jnp.float32)
    out_ref[...] = out + b2_ref[...]


def kernel(x, embed, W1, b1, W2, b2):
    batch = x.shape[0]
    x = x.astype(jnp.int32)
    w1r = W1.reshape(HIDDEN, CONTEXT, EMBED).transpose(1, 2, 0)
    b1_2d = b1.reshape(1, HIDDEN)
    w2t = W2.T
    b2_2d = b2.reshape(1, NOUT)

    tbl = pl.pallas_call(
        _table_body,
        grid=(CONTEXT,),
        in_specs=[
            pl.BlockSpec((VOCAB, EMBED), lambda t: (0, 0)),
            pl.BlockSpec((1, EMBED, HIDDEN), lambda t: (t, 0, 0)),
            pl.BlockSpec((1, HIDDEN), lambda t: (0, 0)),
        ],
        out_specs=pl.BlockSpec((1, VOCAB, HIDDEN), lambda t: (t, 0, 0)),
        out_shape=jax.ShapeDtypeStruct((CONTEXT, VOCAB, HIDDEN), jnp.float32),
    )(embed, w1r, b1_2d)

    out = pl.pallas_call(
        _mlp_body,
        grid=(batch // BB,),
        in_specs=[
            pl.BlockSpec((BB, CONTEXT), lambda i: (i, 0)),
            pl.BlockSpec((CONTEXT, VOCAB, HIDDEN), lambda i: (0, 0, 0)),
            pl.BlockSpec((HIDDEN, NOUT), lambda i: (0, 0)),
            pl.BlockSpec((1, NOUT), lambda i: (0, 0)),
        ],
        out_specs=pl.BlockSpec((BB, NOUT), lambda i: (i, 0)),
        out_shape=jax.ShapeDtypeStruct((batch, NOUT), jnp.float32),
    )(x, tbl, w2t, b2_2d)
    return out
